# Initial kernel scaffold; baseline (speedup 1.0000x reference)
import jax
import jax.numpy as jnp
from jax.experimental import pallas as pl


def kernel(lengths, node_attrs, edge_index, atomic_numbers, r_max):
    raise NotImplementedError("write your pallas kernel here")


# speedup vs baseline: 10.4785x; 10.4785x over previous
import functools

import jax
import jax.numpy as jnp
from jax import lax
from jax.experimental import pallas as pl
from jax.experimental.pallas import tpu as pltpu
from jax.experimental.pallas import tpu_sc as plsc

N_NODES = 10000
N_EDGES = 320000

NC = 2
NS = 16
NW = NC * NS
LANES = 16
WIN = 128
WINS_PER_TILE = 79
EDGES_PER_TILE = WIN * WINS_PER_TILE
E_PAD = NW * EDGES_PER_TILE
PAD_ROWS = 240
N_PAD = N_NODES + PAD_ROWS
SLICE = N_PAD // NS

R_MIN = 0.2
R12_CUTOFF = 0.8
INV_WIDTH = 10.0


def _sc_partials(lengths_r, src_r, dst_r, rmax_v):
    mesh = plsc.VectorSubcoreMesh(core_axis_name="c", subcore_axis_name="s")

    @functools.partial(
        pl.kernel,
        out_type=jax.ShapeDtypeStruct((NC, N_PAD), jnp.float32),
        mesh=mesh,
        scratch_types=[
            pltpu.VMEM((WINS_PER_TILE, WIN), jnp.float32),
            pltpu.VMEM((WINS_PER_TILE, WIN), jnp.float32),
            pltpu.VMEM((WINS_PER_TILE, WIN), jnp.int32),
            pltpu.VMEM((WINS_PER_TILE, WIN), jnp.int32),
            pltpu.VMEM((LANES,), jnp.float32),
            pltpu.VMEM((SLICE,), jnp.float32),
            pltpu.VMEM_SHARED((N_PAD,), jnp.float32),
        ],
    )
    def k(len_hbm, src_hbm, dst_hbm, rmax_hbm, out_hbm,
          len_v, qv, si_v, di_v, rm_v, tmp_v, acc_sh):
        c = lax.axis_index("c")
        s = lax.axis_index("s")
        wid = c * NS + s

        pltpu.sync_copy(len_hbm.at[wid], len_v)
        pltpu.sync_copy(src_hbm.at[wid], si_v)
        pltpu.sync_copy(dst_hbm.at[wid], di_v)
        pltpu.sync_copy(rmax_hbm, rm_v)

        @pl.loop(0, SLICE // LANES)
        def _(i):
            tmp_v[pl.ds(i * LANES, LANES)] = jnp.zeros((LANES,), jnp.float32)

        pltpu.sync_copy(tmp_v, acc_sh.at[pl.ds(s * SLICE, SLICE)])
        plsc.subcore_barrier()

        rmax = rm_v[...]

        @pl.loop(0, WINS_PER_TILE)
        def _(w):
            for ch in range(WIN // LANES):
                l = len_v[w, pl.ds(ch * LANES, LANES)]
                r = jnp.maximum(l, R_MIN)
                x = jnp.clip(r / rmax, 0.0, 1.0)
                y = 1.0 - x
                y2 = y * y
                y3 = y2 * y
                cut = y3 * y3
                r2 = r * r
                r4 = r2 * r2
                r8 = r4 * r4
                r12 = r8 * r4
                v = cut / r12
                t = jnp.clip((R12_CUTOFF - r) * INV_WIDTH, 0.0, 1.0)
                sm = t * t * (3.0 - 2.0 * t)
                qv[w, pl.ds(ch * LANES, LANES)] = v * sm * 0.25

            pltpu.sync_copy(qv.at[w], acc_sh.at[si_v.at[w]], add=True)
            pltpu.sync_copy(qv.at[w], acc_sh.at[di_v.at[w]], add=True)

        plsc.subcore_barrier()
        pltpu.sync_copy(acc_sh.at[pl.ds(s * SLICE, SLICE)],
                        out_hbm.at[c, pl.ds(s * SLICE, SLICE)])

    return k(lengths_r, src_r, dst_r, rmax_v)


def _tc_combine(partials):
    def body(p_ref, o_ref):
        o_ref[...] = p_ref[0, :] + p_ref[1, :]

    return pl.pallas_call(
        body,
        out_shape=jax.ShapeDtypeStruct((N_PAD,), jnp.float32),
    )(partials)


def kernel(lengths, node_attrs, edge_index, atomic_numbers, r_max):
    del node_attrs, atomic_numbers
    pad = E_PAD - N_EDGES
    pad_idx = (N_NODES + jnp.arange(pad, dtype=jnp.int32) % PAD_ROWS)
    lengths_r = jnp.concatenate(
        [lengths.astype(jnp.float32), jnp.ones((pad,), jnp.float32)]
    ).reshape(NW, WINS_PER_TILE, WIN)
    src_r = jnp.concatenate(
        [edge_index[0].astype(jnp.int32), pad_idx]
    ).reshape(NW, WINS_PER_TILE, WIN)
    dst_r = jnp.concatenate(
        [edge_index[1].astype(jnp.int32), pad_idx]
    ).reshape(NW, WINS_PER_TILE, WIN)
    rmax_v = jnp.broadcast_to(r_max.astype(jnp.float32), (LANES,))

    partials = _sc_partials(lengths_r, src_r, dst_r, rmax_v)
    return _tc_combine(partials)[:N_NODES]


# speedup vs baseline: 13.1681x; 1.2567x over previous
import functools

import jax
import jax.numpy as jnp
from jax import lax
from jax.experimental import pallas as pl
from jax.experimental.pallas import tpu as pltpu
from jax.experimental.pallas import tpu_sc as plsc

N_NODES = 10000
N_EDGES = 320000

NC = 2
NS = 16
NW = NC * NS
LANES = 16
WIN = 128
EDGES_PER_TILE = N_EDGES // NW
FULL_WINS = EDGES_PER_TILE // WIN
TAIL = EDGES_PER_TILE - FULL_WINS * WIN
N_PAD = 10240
SLICE = N_PAD // NS

R_MIN = 0.2
R12_CUTOFF = 0.8
INV_WIDTH = 10.0


def _edge_v(l, rmax):
    r = jnp.maximum(l, R_MIN)
    x = jnp.clip(r / rmax, 0.0, 1.0)
    y = 1.0 - x
    y2 = y * y
    y3 = y2 * y
    cut = y3 * y3
    r2 = r * r
    r4 = r2 * r2
    r8 = r4 * r4
    r12 = r8 * r4
    t = jnp.clip((R12_CUTOFF - r) * INV_WIDTH, 0.0, 1.0)
    sm = t * t * (3.0 - 2.0 * t)
    return cut / r12 * sm * 0.25


def _sc_partials(lengths, edge_index, rmax_v):
    mesh = plsc.VectorSubcoreMesh(core_axis_name="c", subcore_axis_name="s")

    @functools.partial(
        pl.kernel,
        out_type=jax.ShapeDtypeStruct((NC, N_PAD), jnp.float32),
        mesh=mesh,
        scratch_types=[
            pltpu.VMEM((EDGES_PER_TILE,), jnp.float32),
            pltpu.VMEM((EDGES_PER_TILE,), jnp.float32),
            pltpu.VMEM((EDGES_PER_TILE,), jnp.int32),
            pltpu.VMEM((EDGES_PER_TILE,), jnp.int32),
            pltpu.VMEM((LANES,), jnp.float32),
            pltpu.VMEM((SLICE,), jnp.float32),
            pltpu.VMEM_SHARED((N_PAD,), jnp.float32),
        ],
    )
    def k(len_hbm, ei_hbm, rmax_hbm, out_hbm,
          len_v, qv, si_v, di_v, rm_v, tmp_v, acc_sh):
        c = lax.axis_index("c")
        s = lax.axis_index("s")
        wid = c * NS + s
        base = wid * EDGES_PER_TILE

        pltpu.sync_copy(len_hbm.at[pl.ds(base, EDGES_PER_TILE)], len_v)
        pltpu.sync_copy(ei_hbm.at[pl.ds(base, EDGES_PER_TILE)], si_v)
        pltpu.sync_copy(ei_hbm.at[pl.ds(N_EDGES + base, EDGES_PER_TILE)], di_v)
        pltpu.sync_copy(rmax_hbm, rm_v)

        @pl.loop(0, SLICE // LANES)
        def _(i):
            tmp_v[pl.ds(i * LANES, LANES)] = jnp.zeros((LANES,), jnp.float32)

        pltpu.sync_copy(tmp_v, acc_sh.at[pl.ds(s * SLICE, SLICE)])
        plsc.subcore_barrier()

        rmax = rm_v[...]

        @pl.loop(0, FULL_WINS)
        def _(w):
            for ch in range(WIN // LANES):
                o = w * WIN + ch * LANES
                qv[pl.ds(o, LANES)] = _edge_v(len_v[pl.ds(o, LANES)], rmax)
            pltpu.sync_copy(qv.at[pl.ds(w * WIN, WIN)],
                            acc_sh.at[si_v.at[pl.ds(w * WIN, WIN)]], add=True)
            pltpu.sync_copy(qv.at[pl.ds(w * WIN, WIN)],
                            acc_sh.at[di_v.at[pl.ds(w * WIN, WIN)]], add=True)

        o = FULL_WINS * WIN
        qv[pl.ds(o, TAIL)] = _edge_v(len_v[pl.ds(o, TAIL)], rmax)
        pltpu.sync_copy(qv.at[pl.ds(o, TAIL)],
                        acc_sh.at[si_v.at[pl.ds(o, TAIL)]], add=True)
        pltpu.sync_copy(qv.at[pl.ds(o, TAIL)],
                        acc_sh.at[di_v.at[pl.ds(o, TAIL)]], add=True)

        plsc.subcore_barrier()
        pltpu.sync_copy(acc_sh.at[pl.ds(s * SLICE, SLICE)],
                        out_hbm.at[c, pl.ds(s * SLICE, SLICE)])

    return k(lengths, edge_index, rmax_v)


def _tc_combine(partials):
    def body(p_ref, o_ref):
        o_ref[...] = p_ref[0, :] + p_ref[1, :]

    return pl.pallas_call(
        body,
        out_shape=jax.ShapeDtypeStruct((N_PAD,), jnp.float32),
    )(partials)


def kernel(lengths, node_attrs, edge_index, atomic_numbers, r_max):
    del node_attrs, atomic_numbers
    rmax_v = jnp.broadcast_to(r_max.astype(jnp.float32), (LANES,))
    partials = _sc_partials(lengths.astype(jnp.float32),
                            edge_index.astype(jnp.int32).reshape(-1), rmax_v)
    return _tc_combine(partials)[:N_NODES]


# speedup vs baseline: 16.1557x; 1.2269x over previous
import functools

import jax
import jax.numpy as jnp
from jax import lax
from jax.experimental import pallas as pl
from jax.experimental.pallas import tpu as pltpu
from jax.experimental.pallas import tpu_sc as plsc

N_NODES = 10000
N_EDGES = 320000

NC = 2
NS = 16
NW = NC * NS
LANES = 16
WIN = 128
CH_PER_WIN = WIN // LANES
EDGES_PER_TILE = N_EDGES // NW
SUP = 10112
WINS = SUP // WIN
N_PAD = 10240
SLICE = N_PAD // NS

R_MIN = 0.2
R12_CUTOFF = 0.8
INV_WIDTH = 10.0


def _edge_v(l, rmax):
    r = jnp.maximum(l, R_MIN)
    x = jnp.clip(r / rmax, 0.0, 1.0)
    y = 1.0 - x
    y2 = y * y
    y3 = y2 * y
    cut = y3 * y3
    r2 = r * r
    r4 = r2 * r2
    r8 = r4 * r4
    r12 = r8 * r4
    t = jnp.clip((R12_CUTOFF - r) * INV_WIDTH, 0.0, 1.0)
    sm = t * t * (3.0 - 2.0 * t)
    return cut / r12 * sm * 0.25


def _sc_partials(lengths, edge_index, rmax_1):
    mesh = plsc.VectorSubcoreMesh(core_axis_name="c", subcore_axis_name="s")

    @functools.partial(
        pl.kernel,
        out_type=jax.ShapeDtypeStruct((NC, N_PAD), jnp.float32),
        mesh=mesh,
        scratch_types=[
            pltpu.VMEM((SUP,), jnp.float32),
            pltpu.VMEM((SUP,), jnp.float32),
            pltpu.VMEM((2, SUP), jnp.int32),
            pltpu.VMEM((SUP,), jnp.int32),
            pltpu.VMEM((SUP,), jnp.int32),
            pltpu.VMEM((LANES,), jnp.float32),
            pltpu.VMEM((SLICE,), jnp.float32),
            pltpu.VMEM_SHARED((N_PAD,), jnp.float32),
            pltpu.SemaphoreType.DMA,
        ],
    )
    def k(len_hbm, ei_hbm, rmax_hbm, out_hbm,
          len_v, qv, sidi_v, si_v, di_v, rm_v, tmp_v, acc_sh, sem):
        c = lax.axis_index("c")
        s = lax.axis_index("s")
        wid = c * NS + s
        base = wid * EDGES_PER_TILE
        off = base % WIN
        start = pl.multiple_of(base - off, WIN)

        din = pltpu.async_copy(len_hbm.at[pl.ds(start, SUP)], len_v, sem)
        dei = pltpu.async_copy(ei_hbm.at[:, pl.ds(start, SUP)], sidi_v, sem)
        pltpu.sync_copy(rmax_hbm, rm_v)

        @pl.loop(0, SLICE // LANES)
        def _(i):
            tmp_v[pl.ds(i * LANES, LANES)] = jnp.zeros((LANES,), jnp.float32)

        pltpu.sync_copy(tmp_v, acc_sh.at[pl.ds(s * SLICE, SLICE)])
        din.wait()
        dei.wait()
        plsc.subcore_barrier()

        @pl.loop(0, SUP // LANES)
        def _(i):
            o = i * LANES
            si_v[pl.ds(o, LANES)] = sidi_v[0, pl.ds(o, LANES)]
            di_v[pl.ds(o, LANES)] = sidi_v[1, pl.ds(o, LANES)]

        rmax = rm_v[...]
        zv = jnp.zeros((LANES,), jnp.float32)
        ch0 = off // LANES

        def compute_window(w):
            for ch in range(CH_PER_WIN):
                o = w * WIN + ch * LANES
                qv[pl.ds(o, LANES)] = _edge_v(len_v[pl.ds(o, LANES)], rmax)

        def fire(w):
            d1 = pltpu.async_copy(
                qv.at[pl.ds(w * WIN, WIN)],
                acc_sh.at[si_v.at[pl.ds(w * WIN, WIN)]], sem, add=True)
            d2 = pltpu.async_copy(
                qv.at[pl.ds(w * WIN, WIN)],
                acc_sh.at[di_v.at[pl.ds(w * WIN, WIN)]], sem, add=True)
            return d1, d2

        compute_window(0)

        @pl.loop(0, ch0)
        def _(ch):
            qv[pl.ds(ch * LANES, LANES)] = zv

        @pl.loop(0, WINS - 1)
        def _(w):
            d1, d2 = fire(w)
            compute_window(w + 1)
            d1.wait()
            d2.wait()

        @pl.loop(ch0 + 1, CH_PER_WIN)
        def _(ch):
            qv[pl.ds((WINS - 1) * WIN + ch * LANES, LANES)] = zv

        d1, d2 = fire(WINS - 1)
        d1.wait()
        d2.wait()

        plsc.subcore_barrier()
        pltpu.sync_copy(acc_sh.at[pl.ds(s * SLICE, SLICE)],
                        out_hbm.at[c, pl.ds(s * SLICE, SLICE)])

    return k(lengths, edge_index, rmax_1)


def _tc_combine(partials):
    def body(p_ref, o_ref):
        o_ref[...] = p_ref[0, :] + p_ref[1, :]

    return pl.pallas_call(
        body,
        out_shape=jax.ShapeDtypeStruct((N_PAD,), jnp.float32),
    )(partials)


def kernel(lengths, node_attrs, edge_index, atomic_numbers, r_max):
    del node_attrs, atomic_numbers
    rmax_v = jnp.broadcast_to(r_max.astype(jnp.float32), (LANES,))
    partials = _sc_partials(lengths.astype(jnp.float32),
                            edge_index.astype(jnp.int32), rmax_v)
    return _tc_combine(partials)[:N_NODES]
